# baseline (speedup 1.0000x reference)
import jax
import jax.numpy as jnp
from jax import lax
from jax.experimental import pallas as pl
from jax.experimental.pallas import tpu as pltpu
from jax.experimental.pallas import tpu_sc as plsc

NUM_EMB = 1000000
DIM = 64
BATCH = 16384

NUM_CORES = 2
NUM_SUBCORES = 16
NUM_WORKERS = NUM_CORES * NUM_SUBCORES
B_PER_W = BATCH // NUM_WORKERS
CHUNKS_PER_W = B_PER_W // 2
GROUPS = B_PER_W // 16


def _body(x_hbm, tab_hbm, out_hbm, xv, idx2, rows, outv, sem):
    wid = lax.axis_index("s") * NUM_CORES + lax.axis_index("c")
    base = wid * B_PER_W

    pltpu.sync_copy(x_hbm.at[pl.ds(base, B_PER_W)], xv)

    def halve(g, _):
        sl = pl.ds(g * 16, 16)
        idx2[sl] = lax.shift_right_logical(xv[sl], 1)
        return _

    lax.fori_loop(0, GROUPS, halve, None)

    pltpu.async_copy(tab_hbm.at[idx2], rows, sem).wait()

    lane = lax.iota(jnp.int32, 16)

    def extract(g, _):
        rv = lane + g * 16
        xv16 = xv[pl.ds(g * 16, 16)]
        src_off = lax.shift_left(xv16 & 1, 6)
        orow = lax.shift_right_logical(rv, 1)
        ocol = lax.shift_left(rv & 1, 6)
        for c in range(DIM):
            vals = plsc.load_gather(rows, [rv, src_off + c])
            plsc.store_scatter(outv, [orow, ocol + c], vals)
        return _

    lax.fori_loop(0, GROUPS, extract, None)

    pltpu.sync_copy(outv, out_hbm.at[pl.ds(wid * CHUNKS_PER_W, CHUNKS_PER_W)])


@jax.jit
def kernel(x, table):
    xi = x.astype(jnp.int32)
    tab2 = table.reshape(NUM_EMB // 2, 2 * DIM)
    mesh = plsc.VectorSubcoreMesh(
        core_axis_name="c", subcore_axis_name="s",
        num_cores=NUM_CORES, num_subcores=NUM_SUBCORES)
    run = pl.kernel(
        _body,
        out_type=jax.ShapeDtypeStruct((BATCH // 2, 2 * DIM), jnp.float32),
        mesh=mesh,
        scratch_types=[
            pltpu.VMEM((B_PER_W,), jnp.int32),
            pltpu.VMEM((B_PER_W,), jnp.int32),
            pltpu.VMEM((B_PER_W, 2 * DIM), jnp.float32),
            pltpu.VMEM((CHUNKS_PER_W, 2 * DIM), jnp.float32),
            pltpu.SemaphoreType.DMA,
        ],
        compiler_params=pltpu.CompilerParams(needs_layout_passes=False),
    )
    out2 = run(xi, tab2)
    return out2.reshape(BATCH, DIM)


# speedup vs baseline: 2.5898x; 2.5898x over previous
import jax
import jax.numpy as jnp
from jax import lax
from jax.experimental import pallas as pl
from jax.experimental.pallas import tpu as pltpu
from jax.experimental.pallas import tpu_sc as plsc

NUM_EMB = 1000000
DIM = 64
BATCH = 16384

NUM_CORES = 2
NUM_SUBCORES = 16
NUM_WORKERS = NUM_CORES * NUM_SUBCORES
B_PER_W = BATCH // NUM_WORKERS
NBUF = 4
GROUPS = B_PER_W // NBUF


def _fire(tabT_hbm, blocks, sems, s, xq):
    q = lax.max(lax.min(lax.shift_right_logical(xq[s], 7),
                        (NUM_EMB - 1) >> 7), 0)
    col = pl.multiple_of(lax.shift_left(q, 7), 128)
    pltpu.async_copy(tabT_hbm.at[:, pl.ds(col, 128)], blocks.at[s], sems[s])


def _body(x_hbm, tabT_hbm, out_hbm, xv, blocks, rowstage, s0, s1, s2, s3):
    sems = (s0, s1, s2, s3)
    wid = lax.axis_index("s") * NUM_CORES + lax.axis_index("c")
    base = wid * B_PER_W

    pltpu.sync_copy(x_hbm.at[pl.ds(base, B_PER_W)], xv.at[pl.ds(0, B_PER_W)])

    xq0 = xv[pl.ds(0, 16)]
    for s in range(NBUF):
        _fire(tabT_hbm, blocks, sems, s, xq0)

    lane = lax.iota(jnp.int32, 16)

    def group(g, _):
        xq = xv[pl.ds(g * NBUF, 16)]
        xqn = xv[pl.ds((g + 1) * NBUF, 16)]
        for s in range(NBUF):
            h = g * NBUF + s
            m = xq[s] & 127
            mv = lax.broadcast(m, (16,))
            pltpu.make_async_copy(
                tabT_hbm.at[:, pl.ds(0, 128)], blocks.at[s], sems[s]).wait()
            for k in range(DIM // 16):
                vals = plsc.load_gather(
                    blocks, [lax.broadcast(s, (16,)), lane + k * 16, mv])
                rowstage[h, pl.ds(k * 16, 16)] = vals
            _fire(tabT_hbm, blocks, sems, s, xqn)
        return _

    lax.fori_loop(0, GROUPS, group, None)

    for s in range(NBUF):
        pltpu.make_async_copy(
            tabT_hbm.at[:, pl.ds(0, 128)], blocks.at[s], sems[s]).wait()

    pltpu.sync_copy(rowstage, out_hbm.at[pl.ds(base, B_PER_W)])


@jax.jit
def kernel(x, table):
    xi = x.astype(jnp.int32)
    tab_t = table.T
    mesh = plsc.VectorSubcoreMesh(
        core_axis_name="c", subcore_axis_name="s",
        num_cores=NUM_CORES, num_subcores=NUM_SUBCORES)
    run = pl.kernel(
        _body,
        out_type=jax.ShapeDtypeStruct((BATCH, 2 * DIM), jnp.float32),
        mesh=mesh,
        scratch_types=[
            pltpu.VMEM((B_PER_W + 16,), jnp.int32),
            pltpu.VMEM((NBUF, DIM, 2 * DIM), jnp.float32),
            pltpu.VMEM((B_PER_W, 2 * DIM), jnp.float32),
            pltpu.SemaphoreType.DMA,
            pltpu.SemaphoreType.DMA,
            pltpu.SemaphoreType.DMA,
            pltpu.SemaphoreType.DMA,
        ],
        compiler_params=pltpu.CompilerParams(
            needs_layout_passes=False, disable_bounds_checks=True),
    )
    out2 = run(xi, tab_t)
    return out2[:, :DIM]


# speedup vs baseline: 2.8885x; 1.1153x over previous
import jax
import jax.numpy as jnp
from jax import lax
from jax.experimental import pallas as pl
from jax.experimental.pallas import tpu as pltpu
from jax.experimental.pallas import tpu_sc as plsc

NUM_EMB = 1000000
DIM = 64
BATCH = 16384

NUM_CORES = 2
NUM_SUBCORES = 16
NUM_WORKERS = NUM_CORES * NUM_SUBCORES
B_PER_W = BATCH // NUM_WORKERS
NBUF = 8
HALF = B_PER_W // 2
GROUPS = HALF // NBUF


def _fire(tabT_hbm, blocks, sems, s, xq):
    q = lax.max(lax.min(lax.shift_right_logical(xq[s], 7),
                        (NUM_EMB - 1) >> 7), 0)
    col = pl.multiple_of(lax.shift_left(q, 7), 128)
    pltpu.async_copy(tabT_hbm.at[:, pl.ds(col, 128)], blocks.at[s], sems[s])


def _body(x_hbm, tabT_hbm, out_hbm, xv, blocks, rowstage,
          s0, s1, s2, s3, s4, s5, s6, s7):
    sems = (s0, s1, s2, s3, s4, s5, s6, s7)
    wid = lax.axis_index("s") * NUM_CORES + lax.axis_index("c")
    base = wid * B_PER_W

    pltpu.sync_copy(x_hbm.at[pl.ds(base, B_PER_W)], xv.at[pl.ds(0, B_PER_W)])

    xq0 = xv[pl.ds(0, 16)]
    for s in range(NBUF):
        _fire(tabT_hbm, blocks, sems, s, xq0)

    lane = lax.iota(jnp.int32, 16)

    def make_group(half):
        def group(g, _):
            hbase = half * HALF + g * NBUF
            xq = xv[pl.ds(hbase, 16)]
            xqn = xv[pl.ds(hbase + NBUF, 16)]
            for s in range(NBUF):
                m = xq[s] & 127
                mv = lax.broadcast(m, (16,))
                pltpu.make_async_copy(
                    tabT_hbm.at[:, pl.ds(0, 128)], blocks.at[s],
                    sems[s]).wait()
                for k in range(DIM // 16):
                    vals = plsc.load_gather(
                        blocks, [lax.broadcast(s, (16,)), lane + k * 16, mv])
                    rowstage[g * NBUF + s, pl.ds(k * 16, 16)] = vals
                _fire(tabT_hbm, blocks, sems, s, xqn)
            return _
        return group

    for half in range(2):
        lax.fori_loop(0, GROUPS, make_group(half), None)
        pltpu.sync_copy(
            rowstage, out_hbm.at[pl.ds(base + half * HALF, HALF)])

    for s in range(NBUF):
        pltpu.make_async_copy(
            tabT_hbm.at[:, pl.ds(0, 128)], blocks.at[s], sems[s]).wait()


@jax.jit
def kernel(x, table):
    xi = x.astype(jnp.int32)
    tab_t = table.T
    mesh = plsc.VectorSubcoreMesh(
        core_axis_name="c", subcore_axis_name="s",
        num_cores=NUM_CORES, num_subcores=NUM_SUBCORES)
    run = pl.kernel(
        _body,
        out_type=jax.ShapeDtypeStruct((BATCH, 2 * DIM), jnp.float32),
        mesh=mesh,
        scratch_types=[
            pltpu.VMEM((B_PER_W + 16,), jnp.int32),
            pltpu.VMEM((NBUF, DIM, 2 * DIM), jnp.float32),
            pltpu.VMEM((HALF, 2 * DIM), jnp.float32),
        ] + [pltpu.SemaphoreType.DMA] * NBUF,
        compiler_params=pltpu.CompilerParams(
            needs_layout_passes=False, disable_bounds_checks=True),
    )
    out2 = run(xi, tab_t)
    return out2[:, :DIM]
